# Initial kernel scaffold; baseline (speedup 1.0000x reference)
import jax
import jax.numpy as jnp
from jax.experimental import pallas as pl


def kernel(mst_probs, n):
    raise NotImplementedError("write your pallas kernel here")


# speedup vs baseline: 88.6296x; 88.6296x over previous
import jax
import jax.numpy as jnp
import numpy as np
from jax.experimental import pallas as pl
from jax.experimental.pallas import tpu as pltpu

_SIZE = 1600000
_COLS = 128
_ROWS = _SIZE // _COLS

_INT_MIN = np.int32(-2147483648)
_BITMASKS = [np.array(1 << b, dtype=np.uint32).view(np.int32).item()
             for b in range(31, -1, -1)]


def _select_body(n_ref, probs_ref, noise_ref, out_ref, skey_ref):
    p = probs_ref[...] + noise_ref[...]
    i = jax.lax.bitcast_convert_type(p, jnp.int32)
    skey = i ^ (jax.lax.shift_right_arithmetic(i, 31) & jnp.int32(0x7FFFFFFF))
    skey_ref[...] = skey

    n = n_ref[0, 0]

    def step(b, prefix_u):
        mask = jnp.left_shift(jnp.int32(1), jnp.int32(31) - b)
        cand_u = prefix_u | mask
        cand_s = cand_u ^ _INT_MIN
        cnt = jnp.sum((skey_ref[...] >= cand_s).astype(jnp.int32))
        return jnp.where(cnt >= n, cand_u, prefix_u)

    prefix_u = jax.lax.fori_loop(0, 32, step, jnp.int32(0))
    thr_s = prefix_u ^ _INT_MIN
    out_ref[...] = (skey_ref[...] >= thr_s).astype(jnp.float32)


def kernel(mst_probs, n):
    noise = jax.random.gumbel(jax.random.key(42), mst_probs.shape,
                              mst_probs.dtype)
    probs2 = mst_probs.reshape(_ROWS, _COLS)
    noise2 = noise.reshape(_ROWS, _COLS)
    n_arr = jnp.asarray(n, jnp.int32).reshape(1, 1)
    out = pl.pallas_call(
        _select_body,
        out_shape=jax.ShapeDtypeStruct((_ROWS, _COLS), jnp.float32),
        in_specs=[
            pl.BlockSpec(memory_space=pltpu.SMEM),
            pl.BlockSpec(memory_space=pltpu.VMEM),
            pl.BlockSpec(memory_space=pltpu.VMEM),
        ],
        out_specs=pl.BlockSpec(memory_space=pltpu.VMEM),
        scratch_shapes=[pltpu.VMEM((_ROWS, _COLS), jnp.int32)],
    )(n_arr, probs2, noise2)
    return out.reshape(_SIZE)
